# Initial kernel scaffold; baseline (speedup 1.0000x reference)
import jax
import jax.numpy as jnp
from jax.experimental import pallas as pl


def kernel(x, edge_index, edge_type, edge_attr, w1, b1, w2, b2, w3, b3):
    raise NotImplementedError("write your pallas kernel here")


# speedup vs baseline: 17.4144x; 17.4144x over previous
import functools

import jax
import jax.numpy as jnp
from jax import lax
from jax.experimental import pallas as pl
from jax.experimental.pallas import tpu as pltpu
from jax.experimental.pallas import tpu_sc as plsc

N = 10000
E = 320000
D = 128
R = 4

NC = 2
NS = 16
NW = NC * NS

CHUNK = 128
EPB = 79
EPT = EPB * CHUNK
E_PAD = NW * EPT
NPAD = 10240
RPT = N // NS

_BLK = 1000


def _mesh():
    return plsc.VectorSubcoreMesh(
        core_axis_name="c", subcore_axis_name="s", num_cores=NC, num_subcores=NS
    )


def _sck0_body(col_h, et_h, gidx_h, degp_h, colv, etv, gv, degv):
    c = lax.axis_index("c")
    s = lax.axis_index("s")
    w = c * NS + s
    pltpu.sync_copy(col_h.at[w], colv)
    pltpu.sync_copy(et_h.at[w], etv)

    zero16 = jnp.zeros((16,), jnp.float32)

    def _z(i, carry):
        degv[pl.ds(i * 16, 16)] = zero16
        return carry

    lax.fori_loop(0, NPAD // 16, _z, 0)

    ones16 = jnp.ones((16,), jnp.float32)

    def _e(k, carry):
        cc = colv[pl.ds(k * 16, 16)]
        tt = etv[pl.ds(k * 16, 16)]
        gv[pl.ds(k * 16, 16)] = tt * N + cc
        plsc.addupdate_scatter(degv, [cc], ones16)
        return carry

    lax.fori_loop(0, EPT // 16, _e, 0)

    pltpu.sync_copy(gv, gidx_h.at[w])
    pltpu.sync_copy(degv, degp_h.at[w])


def _sck0(col2, et2):
    return pl.kernel(
        _sck0_body,
        out_type=(
            jax.ShapeDtypeStruct((NW, EPT), jnp.int32),
            jax.ShapeDtypeStruct((NW, NPAD), jnp.float32),
        ),
        mesh=_mesh(),
        compiler_params=pltpu.CompilerParams(needs_layout_passes=False, use_tc_tiling_on_sc=False),
        scratch_types=[
            pltpu.VMEM((EPT,), jnp.int32),
            pltpu.VMEM((EPT,), jnp.int32),
            pltpu.VMEM((EPT,), jnp.int32),
            pltpu.VMEM((NPAD,), jnp.float32),
        ],
    )(col2, et2)


def _sck_edge_body(hp_h, gidx_h, attr_h, row_h, p_h, gidxv, attrv, rowv, gbuf, acc, sem):
    c = lax.axis_index("c")
    s = lax.axis_index("s")
    w = c * NS + s
    pltpu.sync_copy(gidx_h.at[w], gidxv)
    pltpu.sync_copy(attr_h.at[w], attrv)
    pltpu.sync_copy(row_h.at[w], rowv)

    zero16 = jnp.zeros((16,), jnp.float32)

    def _zrow(i, carry):
        for q in range(D // 16):
            gbuf[i, pl.ds(q * 16, 16)] = zero16
        return carry

    lax.fori_loop(0, CHUNK, _zrow, 0)

    base = s * RPT
    for k in range(4):
        pltpu.sync_copy(gbuf, acc.at[pl.ds(base + k * CHUNK, CHUNK)])
    pltpu.sync_copy(gbuf.at[pl.ds(0, RPT - 4 * CHUNK)],
                    acc.at[pl.ds(base + 4 * CHUNK, RPT - 4 * CHUNK)])
    plsc.subcore_barrier()

    def _chunk(j, carry):
        pltpu.async_copy(hp_h.at[gidxv.at[j]], gbuf, sem).wait()

        def _srow(i, carry2):
            a = plsc.load_gather(attrv, [jnp.full((16,), j * CHUNK + i, jnp.int32)])
            for q in range(D // 16):
                gbuf[i, pl.ds(q * 16, 16)] = gbuf[i, pl.ds(q * 16, 16)] * a
            return carry2

        lax.fori_loop(0, CHUNK, _srow, 0)
        pltpu.sync_copy(gbuf, acc.at[rowv.at[j]], add=True)
        return carry

    lax.fori_loop(0, EPB, _chunk, 0)

    plsc.subcore_barrier()
    pltpu.sync_copy(acc.at[pl.ds(base, RPT)], p_h.at[c, pl.ds(base, RPT)])


def _sck_edge(hp_flat, gidx3, attr2, row3):
    return pl.kernel(
        _sck_edge_body,
        out_type=jax.ShapeDtypeStruct((NC, N, D), jnp.float32),
        mesh=_mesh(),
        compiler_params=pltpu.CompilerParams(needs_layout_passes=False, use_tc_tiling_on_sc=False),
        scratch_types=[
            pltpu.VMEM((EPB, CHUNK), jnp.int32),
            pltpu.VMEM((EPT,), jnp.float32),
            pltpu.VMEM((EPB, CHUNK), jnp.int32),
            pltpu.VMEM((CHUNK, D), jnp.float32),
            pltpu.VMEM_SHARED((N, D), jnp.float32),
            pltpu.SemaphoreType.DMA,
        ],
    )(hp_flat, gidx3, attr2, row3)


def _tck0_body(degp_ref, dis_ref):
    deg = jnp.sum(degp_ref[...], axis=0)
    dis = jnp.where(deg > 0, lax.rsqrt(deg), 0.0)
    dis_ref[...] = dis[:, None]


def _tck0(degp):
    return pl.pallas_call(
        _tck0_body,
        out_shape=jax.ShapeDtypeStruct((NPAD, 1), jnp.float32),
    )(degp)


def _tck1_body(x_ref, dis_ref, w_ref, hp_ref):
    xs = x_ref[...] * dis_ref[...]
    for r in range(R):
        hp_ref[r, :, :] = jnp.dot(xs, w_ref[r], preferred_element_type=jnp.float32)


def _tck1(x, dis, w):
    return pl.pallas_call(
        _tck1_body,
        grid=(N // _BLK,),
        in_specs=[
            pl.BlockSpec((_BLK, D), lambda i: (i, 0)),
            pl.BlockSpec((_BLK, 1), lambda i: (i, 0)),
            pl.BlockSpec((R, D, D), lambda i: (0, 0, 0)),
        ],
        out_specs=pl.BlockSpec((R, _BLK, D), lambda i: (0, i, 0)),
        out_shape=jax.ShapeDtypeStruct((R, N, D), jnp.float32),
    )(x, dis, w)


def _tck_mid_body(p_ref, dis_ref, b_ref, w_ref, z_ref, hp_ref):
    dis = dis_ref[...]
    v = (p_ref[0] + p_ref[1]) * dis + b_ref[...]
    z = jnp.where(v >= 0, v, 0.01 * v)
    z_ref[...] = z
    zs = z * dis
    for r in range(R):
        hp_ref[r, :, :] = jnp.dot(zs, w_ref[r], preferred_element_type=jnp.float32)


def _tck_mid(p, dis, b, w):
    return pl.pallas_call(
        _tck_mid_body,
        grid=(N // _BLK,),
        in_specs=[
            pl.BlockSpec((NC, _BLK, D), lambda i: (0, i, 0)),
            pl.BlockSpec((_BLK, 1), lambda i: (i, 0)),
            pl.BlockSpec((1, D), lambda i: (0, 0)),
            pl.BlockSpec((R, D, D), lambda i: (0, 0, 0)),
        ],
        out_specs=[
            pl.BlockSpec((_BLK, D), lambda i: (i, 0)),
            pl.BlockSpec((R, _BLK, D), lambda i: (0, i, 0)),
        ],
        out_shape=[
            jax.ShapeDtypeStruct((N, D), jnp.float32),
            jax.ShapeDtypeStruct((R, N, D), jnp.float32),
        ],
    )(p, dis, b, w)


def _tck_fin_body(p_ref, dis_ref, b_ref, x_ref, z1_ref, z2_ref, z_ref):
    v = (p_ref[0] + p_ref[1]) * dis_ref[...] + b_ref[...]
    z3 = jnp.where(v >= 0, v, 0.01 * v)
    z_ref[...] = (x_ref[...] + z1_ref[...] + z2_ref[...] + z3) * 0.25


def _tck_fin(p, dis, b, x, z1, z2):
    return pl.pallas_call(
        _tck_fin_body,
        grid=(N // _BLK,),
        in_specs=[
            pl.BlockSpec((NC, _BLK, D), lambda i: (0, i, 0)),
            pl.BlockSpec((_BLK, 1), lambda i: (i, 0)),
            pl.BlockSpec((1, D), lambda i: (0, 0)),
            pl.BlockSpec((_BLK, D), lambda i: (i, 0)),
            pl.BlockSpec((_BLK, D), lambda i: (i, 0)),
            pl.BlockSpec((_BLK, D), lambda i: (i, 0)),
        ],
        out_specs=pl.BlockSpec((_BLK, D), lambda i: (i, 0)),
        out_shape=jax.ShapeDtypeStruct((N, D), jnp.float32),
    )(p, dis, b, x, z1, z2)


def kernel(x, edge_index, edge_type, edge_attr, w1, b1, w2, b2, w3, b3):
    pad = E_PAD - E
    row2 = jnp.concatenate([edge_index[0], jnp.zeros((pad,), jnp.int32)]).reshape(NW, EPT)
    col2 = jnp.concatenate([edge_index[1], jnp.full((pad,), N, jnp.int32)]).reshape(NW, EPT)
    et2 = jnp.concatenate([edge_type, jnp.zeros((pad,), jnp.int32)]).reshape(NW, EPT)
    attr2 = jnp.concatenate([edge_attr, jnp.zeros((pad,), jnp.float32)]).reshape(NW, EPT)

    gidx2, degp = _sck0(col2, et2)
    gidx3 = gidx2.reshape(NW, EPB, CHUNK)
    row3 = row2.reshape(NW, EPB, CHUNK)

    dis = _tck0(degp)

    hp1 = _tck1(x, dis, w1)
    p1 = _sck_edge(hp1.reshape(R * N, D), gidx3, attr2, row3)
    z1, hp2 = _tck_mid(p1, dis, b1.reshape(1, D), w2)
    p2 = _sck_edge(hp2.reshape(R * N, D), gidx3, attr2, row3)
    z2, hp3 = _tck_mid(p2, dis, b2.reshape(1, D), w3)
    p3 = _sck_edge(hp3.reshape(R * N, D), gidx3, attr2, row3)
    z = _tck_fin(p3, dis, b3.reshape(1, D), x, z1, z2)
    return z
